# Initial kernel scaffold; baseline (speedup 1.0000x reference)
import jax
import jax.numpy as jnp
from jax.experimental import pallas as pl


def kernel(x_dense, x_sparse, dense_embed_weight, sparse_embed_weight, col_embed):
    raise NotImplementedError("write your pallas kernel here")


# speedup vs baseline: 1.1373x; 1.1373x over previous
import functools

import jax
import jax.numpy as jnp
from jax import lax
from jax.experimental import pallas as pl
from jax.experimental.pallas import tpu as pltpu
from jax.experimental.pallas import tpu_sc as plsc

B = 16384
DD = 13
DS = 26
NF = DD + DS
H = 32
L = 16

NW = 32
BPW = B // NW
CB = 32
CHUNKS = BPW // CB
GW = 104
NG = (CB * DS) // GW

def _body(x_hbm, idx_hbm, w_hbm, table_hbm, col_hbm, out_hbm,
          idx_v, gath_v, out_v, xv_v, w_v, col_v, sem):
    wid = lax.axis_index("s") * 2 + lax.axis_index("c")

    pltpu.sync_copy(w_hbm, w_v)
    pltpu.sync_copy(col_hbm, col_v)

    @pl.loop(0, CHUNKS)
    def _chunk(ci):
        b0 = wid * BPW + ci * CB
        pltpu.sync_copy(
            idx_hbm.at[pl.ds(pl.multiple_of((b0 * DS) // GW, 8), NG)], idx_v)
        pltpu.sync_copy(x_hbm.at[pl.ds(pl.multiple_of(b0 * DD, 8), CB * DD)],
                        xv_v.at[pl.ds(0, CB * DD)])

        for g in range(NG):
            pltpu.async_copy(
                table_hbm.at[idx_v.at[g]],
                gath_v.at[pl.ds(g * GW, GW)],
                sem,
            )
        for g in range(NG):
            pltpu.make_async_copy(
                table_hbm.at[idx_v.at[g]],
                gath_v.at[pl.ds(g * GW, GW)],
                sem,
            ).wait()

        @pl.loop(0, CB)
        def _row(bi):
            obase = bi * NF
            xrow = xv_v[pl.ds(bi * DD, L)]
            for d in range(DD):
                x = xrow[d]
                for hh in range(0, H, L):
                    w = w_v[d, pl.ds(hh, L)]
                    c = col_v[d, pl.ds(hh, L)]
                    out_v[obase + d, pl.ds(hh, L)] = (
                        jnp.maximum(x * w, 0.0) + c)
            for s in range(DS):
                for hh in range(0, H, L):
                    g = gath_v[bi * DS + s, pl.ds(hh, L)]
                    c = col_v[DD + s, pl.ds(hh, L)]
                    out_v[obase + DD + s, pl.ds(hh, L)] = g + c

        pltpu.sync_copy(
            out_v, out_hbm.at[pl.ds(pl.multiple_of(b0 * NF, 8), CB * NF)])


@functools.cache
def _sc_embed():
    mesh = plsc.VectorSubcoreMesh(core_axis_name="c", subcore_axis_name="s")
    return functools.partial(
        pl.kernel,
        out_type=jax.ShapeDtypeStruct((B * NF, H), jnp.float32),
        mesh=mesh,
        scratch_types=[
            pltpu.VMEM((NG, GW), jnp.int32),
            pltpu.VMEM((CB * DS, H), jnp.float32),
            pltpu.VMEM((CB * NF, H), jnp.float32),
            pltpu.VMEM((CB * DD + L,), jnp.float32),
            pltpu.VMEM((DD, H), jnp.float32),
            pltpu.VMEM((NF, H), jnp.float32),
            pltpu.SemaphoreType.DMA,
        ],
        compiler_params=pltpu.CompilerParams(use_tc_tiling_on_sc=False),
    )(_body)


def kernel(x_dense, x_sparse, dense_embed_weight, sparse_embed_weight,
           col_embed):
    xf = x_dense.reshape(B * DD)
    idx = x_sparse.astype(jnp.int32).reshape(B * DS // GW, GW)
    out = _sc_embed()(xf, idx, dense_embed_weight, sparse_embed_weight,
                      col_embed)
    return out.reshape(B, NF, H)


# speedup vs baseline: 1.1479x; 1.0093x over previous
import functools

import jax
import jax.numpy as jnp
from jax import lax
from jax.experimental import pallas as pl
from jax.experimental.pallas import tpu as pltpu
from jax.experimental.pallas import tpu_sc as plsc

B = 16384
DD = 13
DS = 26
NF = DD + DS
H = 32
L = 16

NW = 32
BPW = B // NW
CB = 32
CHUNKS = BPW // CB


def _body(x_hbm, idx_hbm, w_hbm, table_hbm, col_hbm, out_hbm,
          idx_v, out_v, xv_v, w_v, col_v, sem):
    wid = lax.axis_index("s") * 2 + lax.axis_index("c")

    pltpu.sync_copy(w_hbm, w_v)
    pltpu.sync_copy(col_hbm, col_v)

    @pl.loop(0, CHUNKS)
    def _chunk(ci):
        b0 = pl.multiple_of(wid * BPW + ci * CB, 8)
        pltpu.sync_copy(idx_hbm.at[pl.ds(b0, CB)], idx_v)
        pltpu.sync_copy(x_hbm.at[pl.ds(b0, CB)], xv_v)

        for bi in range(CB):
            pltpu.async_copy(
                table_hbm.at[idx_v.at[bi]],
                out_v.at[bi, pl.ds(DD, DS)],
                sem,
            )
        for bi in range(CB):
            pltpu.make_async_copy(
                table_hbm.at[idx_v.at[bi]],
                out_v.at[bi, pl.ds(DD, DS)],
                sem,
            ).wait()

        @pl.loop(0, CB)
        def _row(bi):
            xrow = xv_v[bi, pl.ds(0, L)]
            for d in range(DD):
                x = xrow[d]
                for hh in range(0, H, L):
                    w = w_v[d, pl.ds(hh, L)]
                    c = col_v[d, pl.ds(hh, L)]
                    out_v[bi, d, pl.ds(hh, L)] = jnp.maximum(x * w, 0.0) + c
            for s in range(DS):
                for hh in range(0, H, L):
                    c = col_v[DD + s, pl.ds(hh, L)]
                    out_v[bi, DD + s, pl.ds(hh, L)] += c

        pltpu.sync_copy(out_v, out_hbm.at[pl.ds(b0, CB)])


@functools.cache
def _sc_embed():
    mesh = plsc.VectorSubcoreMesh(core_axis_name="c", subcore_axis_name="s")
    return functools.partial(
        pl.kernel,
        out_type=jax.ShapeDtypeStruct((B, NF, H), jnp.float32),
        mesh=mesh,
        scratch_types=[
            pltpu.VMEM((CB, DS), jnp.int32),
            pltpu.VMEM((CB, NF, H), jnp.float32),
            pltpu.VMEM((CB, L), jnp.float32),
            pltpu.VMEM((DD, H), jnp.float32),
            pltpu.VMEM((NF, H), jnp.float32),
            pltpu.SemaphoreType.DMA,
        ],
        compiler_params=pltpu.CompilerParams(use_tc_tiling_on_sc=False),
    )(_body)


def kernel(x_dense, x_sparse, dense_embed_weight, sparse_embed_weight,
           col_embed):
    idx = x_sparse.astype(jnp.int32)
    xp = jnp.pad(x_dense, ((0, 0), (0, L - DD)))
    return _sc_embed()(xp, idx, dense_embed_weight,
                       sparse_embed_weight, col_embed)


# speedup vs baseline: 1.2242x; 1.0665x over previous
import functools

import jax
import jax.numpy as jnp
from jax import lax
from jax.experimental import pallas as pl
from jax.experimental.pallas import tpu as pltpu
from jax.experimental.pallas import tpu_sc as plsc

B = 16384
DD = 13
DS = 26
NF = DD + DS
H = 32
L = 16

NW = 32
BPW = B // NW
CBL = 32
CHUNKS = BPW // CBL
NBLK = CBL // L


def _scalar(ref, r, c):
    row = ref[r, pl.ds((c // L) * L, L)]
    return row[c % L]


def _body(x_hbm, idx_hbm, w_hbm, table_hbm, col_hbm, out_hbm,
          idx_v, x_v, gath_v, sout_v, dout_v, w_v, col_v,
          sem_g, sem_o):
    wid = lax.axis_index("s") * 2 + lax.axis_index("c")

    pltpu.sync_copy(w_hbm, w_v)
    pltpu.sync_copy(col_hbm, col_v)
    iota = jax.lax.iota(jnp.int32, L)

    @pl.loop(0, CHUNKS)
    def _chunk(ci):
        p = ci % 2
        c0 = pl.multiple_of(wid * BPW + ci * CBL, 8)

        pltpu.sync_copy(idx_hbm.at[:, pl.ds(c0, CBL)], idx_v)
        pltpu.sync_copy(x_hbm.at[:, pl.ds(c0, CBL)], x_v)

        for s in range(DS):
            pltpu.async_copy(
                table_hbm.at[idx_v.at[s]],
                gath_v.at[pl.ds(s * CBL, CBL)],
                sem_g,
            )

        @pl.when(ci >= 2)
        def _():
            pltpu.make_async_copy(
                dout_v.at[0], out_hbm.at[pl.ds(0, DD), :, pl.ds(0, CBL)],
                sem_o).wait()
            pltpu.make_async_copy(
                sout_v.at[0], out_hbm.at[pl.ds(DD, DS), :, pl.ds(0, CBL)],
                sem_o).wait()

        @pl.loop(0, DD)
        def _dense(f):
            for hh in range(0, H, L):
                wrow = w_v[f, pl.ds(hh, L)]
                crow = col_v[f, pl.ds(hh, L)]
                for j in range(L):
                    h = hh + j
                    w = wrow[j]
                    c = crow[j]
                    for blk in range(NBLK):
                        xb = x_v[f, pl.ds(blk * L, L)]
                        dout_v[p, f, h, pl.ds(blk * L, L)] = (
                            jnp.maximum(xb * w, 0.0) + c)

        pltpu.async_copy(
            dout_v.at[p],
            out_hbm.at[pl.ds(0, DD), :, pl.ds(c0, CBL)],
            sem_o,
        )

        for s in range(DS):
            pltpu.make_async_copy(
                table_hbm.at[idx_v.at[s]],
                gath_v.at[pl.ds(s * CBL, CBL)],
                sem_g,
            ).wait()

        @pl.loop(0, DS)
        def _sparse(s):
            base = s * CBL
            for hh in range(0, H, L):
                crow = col_v[DD + s, pl.ds(hh, L)]
                for j in range(L):
                    h = hh + j
                    c = crow[j]
                    hvec = jnp.full((L,), h, dtype=jnp.int32)
                    for blk in range(NBLK):
                        rows = iota + (base + blk * L)
                        g = plsc.load_gather(gath_v, [rows, hvec])
                        sout_v[p, s, h, pl.ds(blk * L, L)] = g + c

        pltpu.async_copy(
            sout_v.at[p],
            out_hbm.at[pl.ds(DD, DS), :, pl.ds(c0, CBL)],
            sem_o,
        )

    for _ in range(2):
        pltpu.make_async_copy(
            dout_v.at[0], out_hbm.at[pl.ds(0, DD), :, pl.ds(0, CBL)],
            sem_o).wait()
        pltpu.make_async_copy(
            sout_v.at[0], out_hbm.at[pl.ds(DD, DS), :, pl.ds(0, CBL)],
            sem_o).wait()


@functools.cache
def _sc_embed():
    mesh = plsc.VectorSubcoreMesh(core_axis_name="c", subcore_axis_name="s")
    return functools.partial(
        pl.kernel,
        out_type=jax.ShapeDtypeStruct((NF, H, B), jnp.float32),
        mesh=mesh,
        scratch_types=[
            pltpu.VMEM((DS, CBL), jnp.int32),
            pltpu.VMEM((DD, CBL), jnp.float32),
            pltpu.VMEM((DS * CBL, H), jnp.float32),
            pltpu.VMEM((2, DS, H, CBL), jnp.float32),
            pltpu.VMEM((2, DD, H, CBL), jnp.float32),
            pltpu.VMEM((DD, H), jnp.float32),
            pltpu.VMEM((NF, H), jnp.float32),
            pltpu.SemaphoreType.DMA,
            pltpu.SemaphoreType.DMA,
        ],
        compiler_params=pltpu.CompilerParams(
            use_tc_tiling_on_sc=False, needs_layout_passes=False),
    )(_body)


def kernel(x_dense, x_sparse, dense_embed_weight, sparse_embed_weight,
           col_embed):
    xt = x_dense.T
    idxt = x_sparse.astype(jnp.int32).T
    out = _sc_embed()(xt, idxt, dense_embed_weight, sparse_embed_weight,
                      col_embed)
    return jnp.transpose(out, (2, 0, 1))
